# baseline (speedup 1.0000x reference)
import functools

import jax
import jax.numpy as jnp
import numpy as np
from jax.experimental import pallas as pl
from jax.experimental.pallas import tpu as pltpu

H, D, ROPE, TOPK = 16, 128, 64, 512
THETA = 10000.0
NEG_FILL = -1e30
INV_SQRT_D = float(D) ** -0.5


def _rope_cos_sin(seq_len):
    inv_freq = 1.0 / (THETA ** (np.arange(0, ROPE, 2, dtype=np.float64) / ROPE))
    t = np.arange(seq_len, dtype=np.float64)
    ang = np.outer(t, inv_freq)
    return jnp.asarray(np.cos(ang), jnp.float32), jnp.asarray(np.sin(ang), jnp.float32)


def _apply_rope_interleave(x, cos, sin):
    x1 = x[..., 0::2]
    x2 = x[..., 1::2]
    o1 = x1 * cos - x2 * sin
    o2 = x1 * sin + x2 * cos
    return jnp.stack([o1, o2], axis=-1).reshape(x.shape)


def _rope_tables(seq_len):
    inv_freq = 1.0 / (THETA ** (np.arange(0, ROPE, 2, dtype=np.float64) / ROPE))
    t = np.arange(seq_len, dtype=np.float64)
    ang = np.outer(t, inv_freq)
    cos = np.cos(ang).astype(np.float32)
    sin = np.sin(ang).astype(np.float32)
    C = np.empty((seq_len, ROPE), np.float32)
    SG = np.empty((seq_len, ROPE), np.float32)
    C[:, 0::2] = cos
    C[:, 1::2] = cos
    SG[:, 0::2] = -sin
    SG[:, 1::2] = sin
    return jnp.asarray(C), jnp.asarray(SG)


def _swap_pairs(x):
    n = x.shape[-1]
    lane = jax.lax.broadcasted_iota(jnp.int32, x.shape, x.ndim - 1)
    even = (lane & 1) == 0
    return jnp.where(even, pltpu.roll(x, n - 1, x.ndim - 1), pltpu.roll(x, 1, x.ndim - 1))


def _qproj_body(hs_ref, wq_ref, c_ref, sg_ref, out_ref):
    q = jax.lax.dot_general(hs_ref[...], wq_ref[...], (((1,), (0,)), ((), ())),
                            preferred_element_type=jnp.float32)
    c = c_ref[...]
    sg = sg_ref[...]
    rows = q.shape[0]
    pieces = []
    for h in range(H):
        nope = jax.lax.slice(q, (0, h * D), (rows, h * D + D - ROPE))
        qr = jax.lax.slice(q, (0, h * D + D - ROPE), (rows, (h + 1) * D))
        roped = qr * c + _swap_pairs(qr) * sg
        pieces.append(nope.astype(jnp.bfloat16))
        pieces.append(roped.astype(jnp.bfloat16))
    out_ref[...] = jnp.concatenate(pieces, axis=1)


def _sort_stages(n):
    ks, js = [], []
    k = 2
    while k <= n:
        j = k // 2
        while j >= 1:
            ks.append(k)
            js.append(j)
            j //= 2
        k *= 2
    return np.asarray(ks, np.int32), np.asarray(js, np.int32)


def _compare_swap(v, ix, j, k, n):
    lane = jax.lax.broadcasted_iota(jnp.int32, v.shape, v.ndim - 1)
    lower = (lane & j) == 0
    descblk = (lane & k) == 0
    keepmax = lower == descblk
    pv = jnp.where(lower, pltpu.roll(v, n - j, v.ndim - 1), pltpu.roll(v, j, v.ndim - 1))
    pi = jnp.where(lower, pltpu.roll(ix, n - j, v.ndim - 1), pltpu.roll(ix, j, v.ndim - 1))
    gt = (pv > v) | ((pv == v) & (pi < ix))
    take = keepmax == gt
    return jnp.where(take, pv, v), jnp.where(take, pi, ix)


def _main_body(ks_ref, js_ref, q_ref, w_ref, k_ref, vals_ref, idx_ref, *, blk_q, seq):
    pid = pl.program_id(0)
    qb = q_ref[...]
    wf = w_ref[...].astype(jnp.float32)
    kb = k_ref[...]

    acc = jnp.zeros((blk_q, seq), jnp.float32)
    for h in range(H):
        qh = jax.lax.slice(qb, (0, h * D), (blk_q, (h + 1) * D))
        s = jax.lax.dot_general(qh, kb, (((1,), (1,)), ((), ())),
                                preferred_element_type=jnp.float32)
        s = s * INV_SQRT_D
        r = jnp.maximum(s, 0.0).astype(jnp.bfloat16).astype(jnp.float32)
        wh = jax.lax.slice(wf, (0, h), (blk_q, h + 1))
        acc = acc + wh * r

    col = jax.lax.broadcasted_iota(jnp.int32, (blk_q, seq), 1)
    row = pid * blk_q + jax.lax.broadcasted_iota(jnp.int32, (blk_q, seq), 0)
    fill = NEG_FILL * (1.0 + col.astype(jnp.float32) * jnp.float32(2.0 ** -20))
    v = jnp.where(col <= row, acc, fill)

    n_stages = ks_ref.shape[0]

    def stage(s, carry):
        vv, ix = carry
        return _compare_swap(vv, ix, js_ref[s], ks_ref[s], seq)

    v, ix = jax.lax.fori_loop(0, n_stages, stage, (v, col))

    out_v = jax.lax.slice(v, (0, 0), (blk_q, TOPK))
    out_i = jax.lax.slice(ix, (0, 0), (blk_q, TOPK))
    vals_ref[...] = jnp.where(out_v <= -1e29, jnp.float32(NEG_FILL), out_v)
    idx_ref[...] = out_i


def _indexer_topk(qb, wb, k_roped, *, blk_q=256):
    seq = qb.shape[0]
    ks, js = _sort_stages(seq)
    body = functools.partial(_main_body, blk_q=blk_q, seq=seq)
    vals, idx = pl.pallas_call(
        body,
        grid=(seq // blk_q,),
        in_specs=[
            pl.BlockSpec(memory_space=pltpu.SMEM),
            pl.BlockSpec(memory_space=pltpu.SMEM),
            pl.BlockSpec((blk_q, H * D), lambda i: (i, 0)),
            pl.BlockSpec((blk_q, H), lambda i: (i, 0)),
            pl.BlockSpec((seq, D), lambda i: (0, 0)),
        ],
        out_specs=[
            pl.BlockSpec((blk_q, TOPK), lambda i: (i, 0)),
            pl.BlockSpec((blk_q, TOPK), lambda i: (i, 0)),
        ],
        out_shape=[
            jax.ShapeDtypeStruct((seq, TOPK), jnp.float32),
            jax.ShapeDtypeStruct((seq, TOPK), jnp.int32),
        ],
    )(jnp.asarray(ks), jnp.asarray(js), qb, wb, k_roped)
    return vals, idx


def kernel(hidden_states, wq, wk, w_proj):
    b, seq, _ = hidden_states.shape
    cos, sin = _rope_cos_sin(seq)

    k = hidden_states @ wk
    k_nope, k_rope = k[..., : D - ROPE], k[..., D - ROPE:]
    k_rope = _apply_rope_interleave(k_rope, cos[None, :, :], sin[None, :, :])
    k = jnp.concatenate([k_nope, k_rope], axis=-1)
    weights = (hidden_states @ w_proj) * (H ** -0.5)

    q = (hidden_states @ wq).reshape(b, seq, H, D)
    q_nope, q_rope = q[..., : D - ROPE], q[..., D - ROPE:]
    q_rope = _apply_rope_interleave(q_rope, cos[None, :, None, :], sin[None, :, None, :])
    q = jnp.concatenate([q_nope, q_rope], axis=-1)

    vals, idx = _indexer_topk(
        q[0].reshape(seq, H * D).astype(jnp.bfloat16),
        weights[0].astype(jnp.bfloat16), k[0].astype(jnp.bfloat16))
    return vals[None], idx[None]


# speedup vs baseline: 1.9337x; 1.9337x over previous
import functools

import jax
import jax.numpy as jnp
import numpy as np
from jax.experimental import pallas as pl
from jax.experimental.pallas import tpu as pltpu

H, D, ROPE, TOPK = 16, 128, 64, 512
THETA = 10000.0
NEG_FILL = -1e30
INV_SQRT_D = float(D) ** -0.5


def _rope_cos_sin(seq_len):
    inv_freq = 1.0 / (THETA ** (np.arange(0, ROPE, 2, dtype=np.float64) / ROPE))
    t = np.arange(seq_len, dtype=np.float64)
    ang = np.outer(t, inv_freq)
    return jnp.asarray(np.cos(ang), jnp.float32), jnp.asarray(np.sin(ang), jnp.float32)


def _apply_rope_interleave(x, cos, sin):
    x1 = x[..., 0::2]
    x2 = x[..., 1::2]
    o1 = x1 * cos - x2 * sin
    o2 = x1 * sin + x2 * cos
    return jnp.stack([o1, o2], axis=-1).reshape(x.shape)


def _rope_tables(seq_len):
    inv_freq = 1.0 / (THETA ** (np.arange(0, ROPE, 2, dtype=np.float64) / ROPE))
    t = np.arange(seq_len, dtype=np.float64)
    ang = np.outer(t, inv_freq)
    cos = np.cos(ang).astype(np.float32)
    sin = np.sin(ang).astype(np.float32)
    C = np.empty((seq_len, ROPE), np.float32)
    SG = np.empty((seq_len, ROPE), np.float32)
    C[:, 0::2] = cos
    C[:, 1::2] = cos
    SG[:, 0::2] = -sin
    SG[:, 1::2] = sin
    return jnp.asarray(C), jnp.asarray(SG)


def _swap_pairs(x):
    n = x.shape[-1]
    lane = jax.lax.broadcasted_iota(jnp.int32, x.shape, x.ndim - 1)
    even = (lane & 1) == 0
    return jnp.where(even, pltpu.roll(x, n - 1, x.ndim - 1), pltpu.roll(x, 1, x.ndim - 1))


def _qproj_body(hs_ref, wq_ref, c_ref, sg_ref, out_ref):
    q = jax.lax.dot_general(hs_ref[...], wq_ref[...], (((1,), (0,)), ((), ())),
                            preferred_element_type=jnp.float32)
    c = c_ref[...]
    sg = sg_ref[...]
    rows = q.shape[0]
    pieces = []
    for h in range(H):
        nope = jax.lax.slice(q, (0, h * D), (rows, h * D + D - ROPE))
        qr = jax.lax.slice(q, (0, h * D + D - ROPE), (rows, (h + 1) * D))
        roped = qr * c + _swap_pairs(qr) * sg
        pieces.append(nope.astype(jnp.bfloat16))
        pieces.append(roped.astype(jnp.bfloat16))
    out_ref[...] = jnp.concatenate(pieces, axis=1)


def _sort_stages(n):
    ks, js = [], []
    k = 2
    while k <= n:
        j = k // 2
        while j >= 1:
            ks.append(k)
            js.append(j)
            j //= 2
        k *= 2
    return np.asarray(ks, np.int32), np.asarray(js, np.int32)


def _compare_swap(v, ix, j, k, n):
    lane = jax.lax.broadcasted_iota(jnp.int32, v.shape, v.ndim - 1)
    lower = (lane & j) == 0
    descblk = (lane & k) == 0
    keepmax = lower == descblk
    pv = jnp.where(lower, pltpu.roll(v, n - j, v.ndim - 1), pltpu.roll(v, j, v.ndim - 1))
    pi = jnp.where(lower, pltpu.roll(ix, n - j, v.ndim - 1), pltpu.roll(ix, j, v.ndim - 1))
    gt = (pv > v) | ((pv == v) & (pi < ix))
    take = keepmax == gt
    return jnp.where(take, pv, v), jnp.where(take, pi, ix)


def _scores_body(q_ref, w_ref, k_ref, out_ref, *, blk_q, seq):
    pid = pl.program_id(0)
    qb = q_ref[...]
    wf = w_ref[...].astype(jnp.float32)
    kb = k_ref[...]

    acc = jnp.zeros((blk_q, seq), jnp.float32)
    for h in range(H):
        qh = jax.lax.slice(qb, (0, h * D), (blk_q, (h + 1) * D))
        s = jax.lax.dot_general(qh, kb, (((1,), (1,)), ((), ())),
                                preferred_element_type=jnp.float32)
        s = s * INV_SQRT_D
        r = jnp.maximum(s, 0.0).astype(jnp.bfloat16).astype(jnp.float32)
        wh = jax.lax.slice(wf, (0, h), (blk_q, h + 1))
        acc = acc + wh * r

    col = jax.lax.broadcasted_iota(jnp.int32, (blk_q, seq), 1)
    row = pid * blk_q + jax.lax.broadcasted_iota(jnp.int32, (blk_q, seq), 0)
    fill = NEG_FILL * (1.0 + col.astype(jnp.float32) * jnp.float32(2.0 ** -20))
    out_ref[...] = jnp.where(col <= row, acc, fill)


def _sort_body(v_ref, vals_ref, idx_ref, *, blk_q, seq):
    v = v_ref[...]
    ix = jax.lax.broadcasted_iota(jnp.int32, (blk_q, seq), 1)
    ks, js = _sort_stages(seq)
    for s in range(len(ks)):
        v, ix = _compare_swap(v, ix, int(js[s]), int(ks[s]), seq)
    out_v = jax.lax.slice(v, (0, 0), (blk_q, TOPK))
    out_i = jax.lax.slice(ix, (0, 0), (blk_q, TOPK))
    vals_ref[...] = jnp.where(out_v <= -1e29, jnp.float32(NEG_FILL), out_v)
    idx_ref[...] = out_i


def _indexer_topk(qb, wb, k_roped, *, blk_q=256, blk_sort=32):
    seq = qb.shape[0]
    scores = pl.pallas_call(
        functools.partial(_scores_body, blk_q=blk_q, seq=seq),
        grid=(seq // blk_q,),
        in_specs=[
            pl.BlockSpec((blk_q, H * D), lambda i: (i, 0)),
            pl.BlockSpec((blk_q, H), lambda i: (i, 0)),
            pl.BlockSpec((seq, D), lambda i: (0, 0)),
        ],
        out_specs=pl.BlockSpec((blk_q, seq), lambda i: (i, 0)),
        out_shape=jax.ShapeDtypeStruct((seq, seq), jnp.float32),
    )(qb, wb, k_roped)

    vals, idx = pl.pallas_call(
        functools.partial(_sort_body, blk_q=blk_sort, seq=seq),
        grid=(seq // blk_sort,),
        in_specs=[pl.BlockSpec((blk_sort, seq), lambda i: (i, 0))],
        out_specs=[
            pl.BlockSpec((blk_sort, TOPK), lambda i: (i, 0)),
            pl.BlockSpec((blk_sort, TOPK), lambda i: (i, 0)),
        ],
        out_shape=[
            jax.ShapeDtypeStruct((seq, TOPK), jnp.float32),
            jax.ShapeDtypeStruct((seq, TOPK), jnp.int32),
        ],
    )(scores)
    return vals, idx


def kernel(hidden_states, wq, wk, w_proj):
    b, seq, _ = hidden_states.shape
    cos, sin = _rope_cos_sin(seq)

    k = hidden_states @ wk
    k_nope, k_rope = k[..., : D - ROPE], k[..., D - ROPE:]
    k_rope = _apply_rope_interleave(k_rope, cos[None, :, :], sin[None, :, :])
    k = jnp.concatenate([k_nope, k_rope], axis=-1)
    weights = (hidden_states @ w_proj) * (H ** -0.5)

    q = (hidden_states @ wq).reshape(b, seq, H, D)
    q_nope, q_rope = q[..., : D - ROPE], q[..., D - ROPE:]
    q_rope = _apply_rope_interleave(q_rope, cos[None, :, None, :], sin[None, :, None, :])
    q = jnp.concatenate([q_nope, q_rope], axis=-1)

    vals, idx = _indexer_topk(
        q[0].reshape(seq, H * D).astype(jnp.bfloat16),
        weights[0].astype(jnp.bfloat16), k[0].astype(jnp.bfloat16))
    return vals[None], idx[None]
